# Initial kernel scaffold; baseline (speedup 1.0000x reference)
import jax
import jax.numpy as jnp
from jax import lax
from jax.experimental import pallas as pl
from jax.experimental.pallas import tpu as pltpu
from jax.experimental.pallas import tpu_sc as plsc

N = 10000
E = 320000
D = 128
F = 4
NF = N * F

NC = 2
NS = 16
NW = NC * NS
L = 16

EW = E // NW
C = 80
NCHUNK = EW // C
RPS = N // NS
ZR = 125


def _mm_body(h_ref, wl_ref, wr_ref, xl_ref, xr_ref):
    h = h_ref[...]
    xl_ref[...] = jnp.dot(h, wl_ref[...], preferred_element_type=jnp.float32)
    xr_ref[...] = jnp.dot(h, wr_ref[...], preferred_element_type=jnp.float32)


def _tc_linear2(h, wl, wr):
    BM = 1000
    grid = (NF // BM,)
    return pl.pallas_call(
        _mm_body,
        grid=grid,
        in_specs=[
            pl.BlockSpec((BM, D), lambda i: (i, 0)),
            pl.BlockSpec((D, D), lambda i: (0, 0)),
            pl.BlockSpec((D, D), lambda i: (0, 0)),
        ],
        out_specs=[
            pl.BlockSpec((BM, D), lambda i: (i, 0)),
            pl.BlockSpec((BM, D), lambda i: (i, 0)),
        ],
        out_shape=[
            jax.ShapeDtypeStruct((NF, D), jnp.float32),
            jax.ShapeDtypeStruct((NF, D), jnp.float32),
        ],
    )(h, wl, wr)


def _fin_body(num_ref, den_ref, b_ref, out_ref):
    n = num_ref[0] + num_ref[1]
    d = den_ref[0][:, 0:1] + den_ref[1][:, 0:1]
    out_ref[...] = jnp.maximum(n / (d + 1e-16) + b_ref[...], 0.0)


def _tc_finalize(num, den, b):
    BM = 1000
    grid = (NF // BM,)
    return pl.pallas_call(
        _fin_body,
        grid=grid,
        in_specs=[
            pl.BlockSpec((NC, BM, D), lambda i: (0, i, 0)),
            pl.BlockSpec((NC, BM, L), lambda i: (0, i, 0)),
            pl.BlockSpec((1, D), lambda i: (0, 0)),
        ],
        out_specs=pl.BlockSpec((BM, D), lambda i: (i, 0)),
        out_shape=jax.ShapeDtypeStruct((NF, D), jnp.float32),
    )(num, den, b)


def _pass_a_body(xl_hbm, xr_hbm, src_hbm, dst_hbm, att_hbm,
                 logits_hbm, wmax_hbm,
                 si_v, di_v, xlr_v, xrr_v, att_v, log_v, wm_v, sem1, sem2):
    cid = lax.axis_index("c")
    sid = lax.axis_index("s")
    wid = sid * NC + cid

    pltpu.sync_copy(att_hbm, att_v)

    def edge_body(e, m):
        acc = jnp.zeros((L,), jnp.float32)
        for j in range(D // L):
            sl = pl.ds(L * j, L)
            z = xlr_v[e, sl] + xrr_v[e, sl]
            lrelu = jnp.maximum(z, 0.2 * z)
            acc = acc + att_v[sl] * lrelu
        logit = jnp.sum(acc)
        log_v[e] = logit
        return jnp.maximum(m, logit)

    m = jnp.float32(-3.0e38)
    for f in range(F):
        off = jnp.int32(f * N)

        def chunk_body(ci, m, off=off, f=f):
            ebase = wid * EW + ci * C
            pltpu.sync_copy(src_hbm.at[pl.ds(ebase, C)], si_v)
            pltpu.sync_copy(dst_hbm.at[pl.ds(ebase, C)], di_v)
            for i in range(C // L):
                sl = pl.ds(L * i, L)
                si_v[sl] = si_v[sl] + off
                di_v[sl] = di_v[sl] + off
            cp1 = pltpu.async_copy(xl_hbm.at[si_v], xlr_v, sem1)
            cp2 = pltpu.async_copy(xr_hbm.at[di_v], xrr_v, sem2)
            cp1.wait()
            cp2.wait()
            m = lax.fori_loop(0, C, edge_body, m)
            pltpu.sync_copy(log_v, logits_hbm.at[pl.ds(f * E + ebase, C)])
            return m

        m = lax.fori_loop(0, NCHUNK, chunk_body, m)

    wm_v[...] = jnp.full((L,), m, jnp.float32)
    pltpu.sync_copy(wm_v, wmax_hbm.at[wid])


def _sc_pass_a(xl, xr, src, dst, att):
    mesh = plsc.VectorSubcoreMesh(core_axis_name="c", subcore_axis_name="s")
    kern = pl.kernel(
        _pass_a_body,
        out_type=[
            jax.ShapeDtypeStruct((F * E,), jnp.float32),
            jax.ShapeDtypeStruct((NW, L), jnp.float32),
        ],
        mesh=mesh,
        scratch_types=[
            pltpu.VMEM((C,), jnp.int32),
            pltpu.VMEM((C,), jnp.int32),
            pltpu.VMEM((C, D), jnp.float32),
            pltpu.VMEM((C, D), jnp.float32),
            pltpu.VMEM((D,), jnp.float32),
            pltpu.VMEM((C,), jnp.float32),
            pltpu.VMEM((L,), jnp.float32),
            pltpu.SemaphoreType.DMA,
            pltpu.SemaphoreType.DMA,
        ],
    )
    return kern(xl, xr, src, dst, att)


def _pass_b_body(xl_hbm, src_hbm, dst_hbm, logits_hbm, wmax_hbm,
                 num_hbm, den_hbm,
                 si_v, di_v, xlr_v, logw_v, val_v, den_v,
                 wmax_v, znum_v, zden_v, num_sh, den_sh, sem1):
    cid = lax.axis_index("c")
    sid = lax.axis_index("s")
    wid = sid * NC + cid

    pltpu.sync_copy(wmax_hbm, wmax_v)
    m16 = wmax_v[0]
    for r in range(1, NW):
        m16 = jnp.maximum(m16, wmax_v[r])
    gm = jnp.full((L,), jnp.max(m16), jnp.float32)

    lane0 = (lax.iota(jnp.int32, (L,)) == 0).astype(jnp.float32)

    def zero_body(r, _):
        for j in range(D // L):
            znum_v[r, pl.ds(L * j, L)] = jnp.zeros((L,), jnp.float32)
        zden_v[r, :] = jnp.zeros((L,), jnp.float32)
        return 0

    lax.fori_loop(0, ZR, zero_body, 0)

    def edge_body(e, _):
        wb = plsc.load_gather(logw_v, [jnp.full((L,), e, jnp.int32)])
        for j in range(D // L):
            sl = pl.ds(L * j, L)
            val_v[e, sl] = wb * xlr_v[e, sl]
        den_v[e, :] = wb * lane0
        return 0

    for f in range(F):
        off = jnp.int32(f * N)
        for k in range(RPS // ZR):
            rb = sid * RPS + k * ZR
            pltpu.sync_copy(znum_v, num_sh.at[pl.ds(rb, ZR)])
            pltpu.sync_copy(zden_v, den_sh.at[pl.ds(rb, ZR)])
        plsc.subcore_barrier()

        def chunk_body(ci, _, off=off, f=f):
            ebase = wid * EW + ci * C
            pltpu.sync_copy(src_hbm.at[pl.ds(ebase, C)], si_v)
            pltpu.sync_copy(dst_hbm.at[pl.ds(ebase, C)], di_v)
            for i in range(C // L):
                sl = pl.ds(L * i, L)
                si_v[sl] = si_v[sl] + off
            pltpu.sync_copy(logits_hbm.at[pl.ds(f * E + ebase, C)], logw_v)
            for i in range(C // L):
                sl = pl.ds(L * i, L)
                logw_v[sl] = jnp.exp(logw_v[sl] - gm)
            pltpu.async_copy(xl_hbm.at[si_v], xlr_v, sem1).wait()
            lax.fori_loop(0, C, edge_body, 0)
            pltpu.sync_copy(val_v, num_sh.at[di_v], add=True)
            pltpu.sync_copy(den_v, den_sh.at[di_v], add=True)
            return 0

        lax.fori_loop(0, NCHUNK, chunk_body, 0)
        plsc.subcore_barrier()

        for k in range(RPS // ZR):
            rb = sid * RPS + k * ZR
            pltpu.sync_copy(num_sh.at[pl.ds(rb, ZR)],
                            num_hbm.at[cid, f, pl.ds(rb, ZR)])
            pltpu.sync_copy(den_sh.at[pl.ds(rb, ZR)],
                            den_hbm.at[cid, f, pl.ds(rb, ZR)])


def _sc_pass_b(xl, src, dst, logits, wmax):
    mesh = plsc.VectorSubcoreMesh(core_axis_name="c", subcore_axis_name="s")
    kern = pl.kernel(
        _pass_b_body,
        out_type=[
            jax.ShapeDtypeStruct((NC, F, N, D), jnp.float32),
            jax.ShapeDtypeStruct((NC, F, N, L), jnp.float32),
        ],
        mesh=mesh,
        scratch_types=[
            pltpu.VMEM((C,), jnp.int32),
            pltpu.VMEM((C,), jnp.int32),
            pltpu.VMEM((C, D), jnp.float32),
            pltpu.VMEM((C,), jnp.float32),
            pltpu.VMEM((C, D), jnp.float32),
            pltpu.VMEM((C, L), jnp.float32),
            pltpu.VMEM((NW, L), jnp.float32),
            pltpu.VMEM((ZR, D), jnp.float32),
            pltpu.VMEM((ZR, L), jnp.float32),
            pltpu.VMEM_SHARED((N, D), jnp.float32),
            pltpu.VMEM_SHARED((N, L), jnp.float32),
            pltpu.SemaphoreType.DMA,
        ],
    )
    return kern(xl, src, dst, logits, wmax)


def kernel(x, edge_index, Wl1, Wr1, att1, b1, Wl2, Wr2, att2, b2):
    src = edge_index[0]
    dst = edge_index[1]
    h = x.reshape(NF, D)
    for (Wl, Wr, att, b) in ((Wl1, Wr1, att1, b1), (Wl2, Wr2, att2, b2)):
        xl, xr = _tc_linear2(h, Wl, Wr)
        logits, wmax = _sc_pass_a(xl, xr, src, dst, att)
        num, den = _sc_pass_b(xl, src, dst, logits, wmax)
        h = _tc_finalize(num.reshape(NC, NF, D), den.reshape(NC, NF, L),
                         b.reshape(1, D))
    return h.reshape(F, N, D)


# speedup vs baseline: 6.3238x; 6.3238x over previous
import jax
import jax.numpy as jnp
from jax import lax
from jax.experimental import pallas as pl
from jax.experimental.pallas import tpu as pltpu
from jax.experimental.pallas import tpu_sc as plsc

N = 10000
E = 320000
D = 128
F = 4
NF = N * F

NC = 2
NS = 16
NW = NC * NS
L = 16

EW = E // NW
C = 80
NCHUNK = EW // C
NP = 10240
NPS = NP // NS
ZR = 32


def _mm_body(h_ref, wl_ref, wr_ref, xl_ref, xr_ref):
    h = h_ref[...]
    xl_ref[...] = jnp.dot(h, wl_ref[...], preferred_element_type=jnp.float32)
    xr_ref[...] = jnp.dot(h, wr_ref[...], preferred_element_type=jnp.float32)


def _tc_linear2(h, wl, wr):
    BM = 1000
    grid = (NF // BM,)
    return pl.pallas_call(
        _mm_body,
        grid=grid,
        in_specs=[
            pl.BlockSpec((BM, D), lambda i: (i, 0)),
            pl.BlockSpec((D, D), lambda i: (0, 0)),
            pl.BlockSpec((D, D), lambda i: (0, 0)),
        ],
        out_specs=[
            pl.BlockSpec((BM, D), lambda i: (i, 0)),
            pl.BlockSpec((BM, D), lambda i: (i, 0)),
        ],
        out_shape=[
            jax.ShapeDtypeStruct((NF, D), jnp.float32),
            jax.ShapeDtypeStruct((NF, D), jnp.float32),
        ],
    )(h, wl, wr)


def _fin_body(num_ref, den_ref, b_ref, out_ref):
    n = num_ref[0] + num_ref[1]
    d = den_ref[0] + den_ref[1]
    out_ref[...] = jnp.maximum(n / (d + 1e-16) + b_ref[...], 0.0)


def _tc_finalize(num, den, b):
    BM = 1000
    grid = (NF // BM,)
    return pl.pallas_call(
        _fin_body,
        grid=grid,
        in_specs=[
            pl.BlockSpec((NC, BM, D), lambda i: (0, i, 0)),
            pl.BlockSpec((NC, BM, 1), lambda i: (0, i, 0)),
            pl.BlockSpec((1, D), lambda i: (0, 0)),
        ],
        out_specs=pl.BlockSpec((BM, D), lambda i: (i, 0)),
        out_shape=jax.ShapeDtypeStruct((NF, D), jnp.float32),
    )(num, den, b)


def _shuffle(v, idx):
    return v.at[idx].get(mode="promise_in_bounds")


def _hsum(v, iota):
    for s in (8, 4, 2, 1):
        v = v + _shuffle(v, iota ^ s)
    return v


def _hmax(v, iota):
    for s in (8, 4, 2, 1):
        v = jnp.maximum(v, _shuffle(v, iota ^ s))
    return v


def _pass_a_body(xl_hbm, xr_hbm, src_hbm, dst_hbm, att_hbm,
                 logits_hbm, wmax_hbm,
                 si_v, di_v, xlr_v, xrr_v, att_v, log_v, wm_v,
                 sem1, sem2):
    cid = lax.axis_index("c")
    sid = lax.axis_index("s")
    wid = sid * NC + cid

    pltpu.sync_copy(att_hbm, att_v)
    iota = lax.iota(jnp.int32, L)

    def group_body(g, m):
        eb = g * L
        logit = jnp.zeros((L,), jnp.float32)
        for u in range(L):
            e = eb + u
            acc = jnp.zeros((L,), jnp.float32)
            for j in range(D // L):
                sl = pl.ds(L * j, L)
                z = xlr_v[e, sl] + xrr_v[e, sl]
                lrelu = jnp.maximum(z, 0.2 * z)
                acc = acc + att_v[sl] * lrelu
            logit = jnp.where(iota == u, _hsum(acc, iota), logit)
        log_v[pl.ds(eb, L)] = logit
        return jnp.maximum(m, logit)

    m = jnp.full((L,), -3.0e38, jnp.float32)
    for f in range(F):
        off = jnp.int32(f * N)

        def chunk_body(ci, m, off=off, f=f):
            ebase = wid * EW + ci * C
            pltpu.sync_copy(src_hbm.at[pl.ds(ebase, C)], si_v)
            pltpu.sync_copy(dst_hbm.at[pl.ds(ebase, C)], di_v)
            for i in range(C // L):
                sl = pl.ds(L * i, L)
                si_v[sl] = si_v[sl] + off
                di_v[sl] = di_v[sl] + off
            cp1 = pltpu.async_copy(xl_hbm.at[si_v], xlr_v, sem1)
            cp2 = pltpu.async_copy(xr_hbm.at[di_v], xrr_v, sem2)
            cp1.wait()
            cp2.wait()
            m = lax.fori_loop(0, C // L, group_body, m)
            pltpu.sync_copy(log_v, logits_hbm.at[pl.ds(f * E + ebase, C)])
            return m

        m = lax.fori_loop(0, NCHUNK, chunk_body, m)

    wm_v[...] = _hmax(m, iota)
    pltpu.sync_copy(wm_v, wmax_hbm.at[wid])


def _sc_pass_a(xl, xr, src, dst, att):
    mesh = plsc.VectorSubcoreMesh(core_axis_name="c", subcore_axis_name="s")
    kern = pl.kernel(
        _pass_a_body,
        out_type=[
            jax.ShapeDtypeStruct((F * E,), jnp.float32),
            jax.ShapeDtypeStruct((NW, L), jnp.float32),
        ],
        mesh=mesh,
        scratch_types=[
            pltpu.VMEM((C,), jnp.int32),
            pltpu.VMEM((C,), jnp.int32),
            pltpu.VMEM((C, D), jnp.float32),
            pltpu.VMEM((C, D), jnp.float32),
            pltpu.VMEM((D,), jnp.float32),
            pltpu.VMEM((C,), jnp.float32),
            pltpu.VMEM((L,), jnp.float32),
            pltpu.SemaphoreType.DMA,
            pltpu.SemaphoreType.DMA,
        ],
    )
    return kern(xl, xr, src, dst, att)


DR = N // D + 1
ZDR = 80


def _pass_b_body(xl_hbm, src_hbm, dst_hbm, logits_hbm, wmax_hbm,
                 num_hbm, den_hbm,
                 si_v, di_v, dr_v, xlr_v, logw_v, den_v,
                 wmax_v, znum_v, zden_v, num_sh, den_sh, sem1):
    cid = lax.axis_index("c")
    sid = lax.axis_index("s")
    wid = sid * NC + cid

    pltpu.sync_copy(wmax_hbm, wmax_v)
    gm = wmax_v[pl.ds(0, L)]
    for r in range(1, NW):
        gm = jnp.maximum(gm, wmax_v[pl.ds(r * L, L)])

    iota = lax.iota(jnp.int32, L)

    def zero_body(r, _):
        for j in range(D // L):
            znum_v[r, pl.ds(L * j, L)] = jnp.zeros((L,), jnp.float32)
        return 0

    lax.fori_loop(0, ZR, zero_body, 0)

    def zden_body(r, _):
        for j in range(D // L):
            zden_v[r, pl.ds(L * j, L)] = jnp.zeros((L,), jnp.float32)
        return 0

    lax.fori_loop(0, ZDR, zden_body, 0)

    def edge_body(g, _):
        eb = g * L
        w16 = logw_v[pl.ds(eb, L)]
        d16 = di_v[pl.ds(eb, L)]
        for u in range(L):
            e = eb + u
            wb = _shuffle(w16, iota * 0 + u)
            for j in range(D // L):
                sl = pl.ds(L * j, L)
                xlr_v[e, sl] = wb * xlr_v[e, sl]
            d = d16[u]
            slot = iota * 0 + (d & (D - 1))
            for j in range(D // L):
                sl = pl.ds(L * j, L)
                den_v[e, sl] = jnp.where(iota == slot - L * j, wb, 0.0)
        return 0

    for f in range(F):
        off = jnp.int32(f * N)
        for k in range(NPS // ZR):
            rb = sid * NPS + k * ZR
            pltpu.sync_copy(znum_v, num_sh.at[pl.ds(rb, ZR)])
        @pl.when(sid == 0)
        def _():
            pltpu.sync_copy(zden_v, den_sh)
        plsc.subcore_barrier()

        def chunk_body(ci, _, off=off, f=f):
            ebase = wid * EW + ci * C
            pltpu.sync_copy(src_hbm.at[pl.ds(ebase, C)], si_v)
            pltpu.sync_copy(dst_hbm.at[pl.ds(ebase, C)], di_v)
            for i in range(C // L):
                sl = pl.ds(L * i, L)
                si_v[sl] = si_v[sl] + off
                dr_v[sl] = lax.shift_right_logical(di_v[sl], 7)
            pltpu.sync_copy(logits_hbm.at[pl.ds(f * E + ebase, C)], logw_v)
            for i in range(C // L):
                sl = pl.ds(L * i, L)
                logw_v[sl] = jnp.exp(logw_v[sl] - gm)
            pltpu.async_copy(xl_hbm.at[si_v], xlr_v, sem1).wait()
            lax.fori_loop(0, C // L, edge_body, 0)
            pltpu.sync_copy(xlr_v, num_sh.at[di_v], add=True)
            pltpu.sync_copy(den_v, den_sh.at[dr_v], add=True)
            return 0

        lax.fori_loop(0, NCHUNK, chunk_body, 0)
        plsc.subcore_barrier()

        for k in range(NPS // ZR):
            rb = sid * NPS + k * ZR
            pltpu.sync_copy(num_sh.at[pl.ds(rb, ZR)],
                            num_hbm.at[cid, f, pl.ds(rb, ZR)])
        @pl.when(sid == 0)
        def _():
            pltpu.sync_copy(den_sh, den_hbm.at[cid, f])
        plsc.subcore_barrier()


def _sc_pass_b(xl, src, dst, logits, wmax):
    mesh = plsc.VectorSubcoreMesh(core_axis_name="c", subcore_axis_name="s")
    kern = pl.kernel(
        _pass_b_body,
        out_type=[
            jax.ShapeDtypeStruct((NC, F, NP, D), jnp.float32),
            jax.ShapeDtypeStruct((NC, F, ZDR, D), jnp.float32),
        ],
        mesh=mesh,
        scratch_types=[
            pltpu.VMEM((C,), jnp.int32),
            pltpu.VMEM((C,), jnp.int32),
            pltpu.VMEM((C,), jnp.int32),
            pltpu.VMEM((C, D), jnp.float32),
            pltpu.VMEM((C,), jnp.float32),
            pltpu.VMEM((C, D), jnp.float32),
            pltpu.VMEM((NW * L,), jnp.float32),
            pltpu.VMEM((ZR, D), jnp.float32),
            pltpu.VMEM((ZDR, D), jnp.float32),
            pltpu.VMEM_SHARED((NP, D), jnp.float32),
            pltpu.VMEM_SHARED((ZDR, D), jnp.float32),
            pltpu.SemaphoreType.DMA,
        ],
    )
    return kern(xl, src, dst, logits, wmax)


def kernel(x, edge_index, Wl1, Wr1, att1, b1, Wl2, Wr2, att2, b2):
    src = edge_index[0]
    dst = edge_index[1]
    h = x.reshape(NF, D)
    for (Wl, Wr, att, b) in ((Wl1, Wr1, att1, b1), (Wl2, Wr2, att2, b2)):
        xl, xr = _tc_linear2(h, Wl, Wr)
        logits, wmax = _sc_pass_a(xl, xr, src, dst, att)
        num, den = _sc_pass_b(xl, src, dst, logits, wmax.reshape(NW * L))
        h = _tc_finalize(num[:, :, :N].reshape(NC, NF, D),
                         den.reshape(NC, F, ZDR * D)[:, :, :N]
                            .reshape(NC, NF, 1),
                         b.reshape(1, D))
    return h.reshape(F, N, D)
